# baseline (speedup 1.0000x reference)
import functools

import jax
import jax.numpy as jnp
import numpy as np
from jax import lax
from jax.experimental import pallas as pl
from jax.experimental.pallas import tpu as pltpu
from jax.experimental.pallas import tpu_sc as plsc

_NUM_HASHES = 5
_BITS = 12
_TABLE = 2048
_M = 3
_U = 0.83
_ROW_LEN = 64
_D = 512
_K = 4096
_QN = 8192
_N = _QN + _K
_HPAD = 8

_W_np = np.zeros((_NUM_HASHES * _BITS + 4, _HPAD), np.float32)
for _h in range(_NUM_HASHES):
    for _j in range(_BITS):
        _W_np[_h * _BITS + _j, _h] = float(2 ** _j) if _j < 11 else 0.0


_QT = _QN // 512
_KT = _K // 512


def _pack(x, at_ref, w_ref, out_ref):
    proj = jnp.dot(x, at_ref[...], preferred_element_type=jnp.float32)
    bits = (proj > 0).astype(jnp.float32)
    out_ref[...] = jnp.dot(bits, w_ref[...], preferred_element_type=jnp.float32)


def _codes_body(q_ref, k_ref, at_ref, w_ref, out_ref, maxsq_ref):
    i = pl.program_id(0)

    @pl.when(i < _KT)
    def _():
        k = k_ref[...]
        mx = jnp.max(jnp.sum(k * k, axis=1))

        @pl.when(i == 0)
        def _():
            maxsq_ref[0] = mx

        @pl.when(i > 0)
        def _():
            maxsq_ref[0] = jnp.maximum(maxsq_ref[0], mx)

    @pl.when((i >= _KT) & (i < 2 * _KT))
    def _():
        scale = _U / jnp.sqrt(maxsq_ref[0])
        ku = k_ref[...] * scale
        sq = jnp.sum(ku * ku, axis=1, keepdims=True)
        s2 = sq * sq
        s4 = s2 * s2
        x = jnp.concatenate([ku, sq, s2, s4], axis=1)
        _pack(x, at_ref, w_ref, out_ref)

    @pl.when(i >= 2 * _KT)
    def _():
        q = q_ref[...]
        nrm = jnp.sqrt(jnp.sum(q * q, axis=1, keepdims=True))
        qn = q / (nrm + 1e-8)
        half = jnp.full((q.shape[0], _M), 0.5, jnp.float32)
        x = jnp.concatenate([qn, half], axis=1)
        _pack(x, at_ref, w_ref, out_ref)


def _codes_call(queries, kernels, at, w):
    def qmap(i):
        return (lax.max(i - 2 * _KT, 0), 0)

    def kmap(i):
        return (jnp.clip(jnp.where(i < _KT, i, i - _KT), 0, _KT - 1), 0)

    def omap(i):
        return (jnp.where(i < _KT, _QT,
                jnp.where(i < 2 * _KT, _QT + (i - _KT), i - 2 * _KT)), 0)

    return pl.pallas_call(
        _codes_body,
        grid=(2 * _KT + _QT,),
        in_specs=[
            pl.BlockSpec((512, _D), qmap),
            pl.BlockSpec((512, _D), kmap),
            pl.BlockSpec((_D + _M, 64), lambda i: (0, 0)),
            pl.BlockSpec((64, _HPAD), lambda i: (0, 0)),
        ],
        out_specs=pl.BlockSpec((512, _HPAD), omap),
        out_shape=jax.ShapeDtypeStruct((_N, _HPAD), jnp.float32),
        scratch_shapes=[pltpu.SMEM((1,), jnp.float32)],
    )(queries, kernels, at, w)


def _vote_body(qc_ref, kc3_ref, rows_ref):
    h = pl.program_id(0)
    qc = qc_ref[...]
    sel = (lax.broadcasted_iota(jnp.int32, (1, _HPAD), 1) == h).astype(jnp.float32)
    qcol = jnp.sum(qc * sel, axis=1, keepdims=True)
    hi = jnp.floor(qcol * (1.0 / 128.0))
    lo = qcol - hi * 128.0
    i16 = lax.broadcasted_iota(jnp.int32, (1, 16), 1).astype(jnp.float32)
    i128 = lax.broadcasted_iota(jnp.int32, (1, 128), 1).astype(jnp.float32)
    ehi = (hi == i16).astype(jnp.float32)
    elo = (lo == i128).astype(jnp.float32)
    counts = lax.dot_general(ehi, elo, (((0,), (0,)), ((), ())),
                             preferred_element_type=jnp.float32)
    maxv = jnp.max(counts)
    tids = (lax.broadcasted_iota(jnp.int32, (16, 128), 0) * 128
            + lax.broadcasted_iota(jnp.int32, (16, 128), 1)).astype(jnp.float32)
    best = jnp.min(jnp.where(counts == maxv, tids, 4096.0))
    kcv = kc3_ref[0]
    match = (kcv == best).astype(jnp.float32)
    tri = (lax.broadcasted_iota(jnp.int32, (128, 128), 0)
           <= lax.broadcasted_iota(jnp.int32, (128, 128), 1)).astype(jnp.float32)
    cumj = lax.dot_general(match, tri, (((1,), (0,)), ((), ())),
                           preferred_element_type=jnp.float32)
    ltri = (lax.broadcasted_iota(jnp.int32, (32, 32), 1)
            < lax.broadcasted_iota(jnp.int32, (32, 32), 0)).astype(jnp.float32)
    p = lax.dot_general(ltri, match, (((1,), (0,)), ((), ())),
                        preferred_element_type=jnp.float32)
    offs = jnp.sum(p, axis=1, keepdims=True)
    cum2 = cumj + offs
    iota64 = lax.broadcasted_iota(jnp.int32, (1, _ROW_LEN), 1)
    rows_f = jnp.zeros((1, _ROW_LEN), jnp.float32)
    for s in range(_ROW_LEN):
        cnt = jnp.sum((cum2 <= float(s)).astype(jnp.float32))
        rows_f = rows_f + cnt * (iota64 == s).astype(jnp.float32)
    rows_ref[0] = rows_f.astype(jnp.int32)


def _vote_call(qc, kc3):
    return pl.pallas_call(
        _vote_body,
        grid=(_NUM_HASHES,),
        in_specs=[
            pl.BlockSpec((_QN, _HPAD), lambda h: (0, 0)),
            pl.BlockSpec((1, 32, 128), lambda h: (h, 0, 0)),
        ],
        out_specs=pl.BlockSpec((1, 1, _ROW_LEN), lambda h: (h, 0, 0)),
        out_shape=jax.ShapeDtypeStruct((_NUM_HASHES, 1, _ROW_LEN), jnp.int32),
    )(qc, kc3)


def _sc_vote_gather(qcT, kcT, kernels):
    mesh = plsc.VectorSubcoreMesh(core_axis_name="c", subcore_axis_name="s")
    nact = _NUM_HASHES * _ROW_LEN

    @functools.partial(
        pl.kernel,
        out_type=jax.ShapeDtypeStruct((nact, _D), jnp.float32),
        mesh=mesh,
        compiler_params=pltpu.CompilerParams(needs_layout_passes=False),
        scratch_types=[
            pltpu.VMEM((6144,), jnp.int32),
            pltpu.VMEM((1536,), jnp.float32),
            pltpu.VMEM((_K,), jnp.float32),
            pltpu.VMEM((16, 128), jnp.int32),
            pltpu.VMEM((16,), jnp.int32),
            pltpu.VMEM((256,), jnp.int32),
            pltpu.VMEM((_ROW_LEN,), jnp.int32),
            pltpu.VMEM((_ROW_LEN, _D), jnp.float32),
            pltpu.VMEM_SHARED((16, 6144), jnp.int32),
            pltpu.VMEM_SHARED((256,), jnp.int32),
            pltpu.SemaphoreType.DMA,
        ],
    )
    def body(qcT_hbm, kcT_hbm, kern_hbm, active_hbm,
             hist_v, qbuf, kbuf, abuf, pbuf, bbuf, rows_v, rbuf,
             hist_sh, best_sh, sem):
        cid = lax.axis_index("c")
        sid = lax.axis_index("s")
        nh = jnp.where(cid == 0, 3, 2)
        hbase = jnp.where(cid == 0, 0, 3)

        z16 = jnp.zeros((16,), jnp.int32)
        zf16 = jnp.zeros((16,), jnp.float32)

        def zb(r, c):
            hist_v[pl.ds(r * 16, 16)] = z16
            return c
        lax.fori_loop(0, 384, zb, 0, unroll=8)

        def ql(j, c):
            pltpu.sync_copy(qcT_hbm.at[hbase + j, pl.ds(sid * 512, 512)],
                            qbuf.at[pl.ds(j * 512, 512)])
            return c
        lax.fori_loop(0, nh, ql, 0)

        ones16 = jnp.ones((16,), jnp.int32)

        def hu(t, c):
            @pl.when(t < nh * 32)
            def _():
                j = t // 32
                code = qbuf[pl.ds(t * 16, 16)]
                v = code.astype(jnp.int32) + j * 2048
                plsc.addupdate_scatter(hist_v, [v], ones16)
            return c
        lax.fori_loop(0, 96, hu, 0, unroll=4)

        pltpu.sync_copy(hist_v, hist_sh.at[sid])
        plsc.subcore_barrier()

        lane = lax.iota(jnp.int32, 16)

        def amh(j, pub):
            pltpu.sync_copy(
                hist_sh.at[:, pl.ds(j * 2048 + sid * 128, 128)], abuf)

            def am(r, carry):
                bv, bi = carry

                def acc_t(t, a):
                    return a + abuf[t, pl.ds(r * 16, 16)]
                cv = lax.fori_loop(1, 16, acc_t, abuf[0, pl.ds(r * 16, 16)],
                                   unroll=16)
                m = jnp.max(cv)
                f = plsc.all_reduce_ffs(cv == jnp.full((16,), m, jnp.int32))
                fi = jnp.max(f)
                upd = m > bv
                return (jnp.where(upd, m, bv),
                        jnp.where(upd, sid * 128 + r * 16 + fi, bi))
            bv, bi = lax.fori_loop(0, 8, am, (jnp.int32(-1), jnp.int32(0)))
            pub = jnp.where(lane == 2 * j, jnp.full((16,), bv, jnp.int32), pub)
            pub = jnp.where(lane == 2 * j + 1,
                            jnp.full((16,), bi, jnp.int32), pub)
            return pub
        pub = lax.fori_loop(0, nh, amh, z16)
        pbuf[...] = pub
        pltpu.sync_copy(pbuf, best_sh.at[pl.ds(sid * 16, 16)])
        plsc.subcore_barrier()

        @pl.when(sid < nh)
        def _():
            h = hbase + sid
            pltpu.sync_copy(best_sh, bbuf)
            neg = jnp.full((16,), -(2 ** 31) + 1, jnp.int32)

            def rd(t, carry):
                bv, bi = carry
                row = bbuf[pl.ds(t * 16, 16)]
                v = jnp.max(jnp.where(lane == 2 * sid, row, neg))
                i = jnp.max(jnp.where(lane == 2 * sid + 1, row, neg))
                upd = v > bv
                return (jnp.where(upd, v, bv), jnp.where(upd, i, bi))
            _, besti = lax.fori_loop(0, 16, rd,
                                     (jnp.int32(-1), jnp.int32(0)), unroll=16)

            pltpu.sync_copy(kcT_hbm.at[h], kbuf)

            def rz(r, c):
                rows_v[pl.ds(r * 16, 16)] = z16
                return c
            lax.fori_loop(0, _ROW_LEN // 16, rz, 0)

            bvec = jnp.full((16,), besti, jnp.int32)

            def cp(t, cnt0):
                code = kbuf[pl.ds(t * 16, 16)].astype(jnp.int32)
                m = code == bvec
                nm = jnp.max(plsc.all_reduce_population_count(m))

                @pl.when(nm > 0)
                def _():
                    cum = plsc.cumsum(m.astype(jnp.int32))
                    pos = cum + (cnt0 - 1)
                    wm = m & (pos < _ROW_LEN)
                    posc = jnp.clip(pos, 0, _ROW_LEN - 1)
                    rowid = lax.iota(jnp.int32, 16) + t * 16
                    plsc.store_scatter(rows_v, [posc], rowid, mask=wm)
                return cnt0 + nm
            total = lax.fori_loop(0, _K // 16, cp, jnp.int32(0), unroll=2)

            pltpu.async_copy(kern_hbm.at[rows_v], rbuf, sem).wait()

            def zz(s, c):
                @pl.when(s >= total)
                def _():
                    def zc(q, cc):
                        rbuf[s, pl.ds(q * 16, 16)] = zf16
                        return cc
                    lax.fori_loop(0, _D // 16, zc, 0, unroll=8)
                return c
            lax.fori_loop(0, _ROW_LEN, zz, 0)
            pltpu.sync_copy(rbuf, active_hbm.at[pl.ds(h * _ROW_LEN, _ROW_LEN)])

    return body(qcT, kcT, kernels)


def _score_body(q_ref, act_ref, out_ref, at_ref):
    @pl.when(pl.program_id(0) == 0)
    def _():
        at_ref[...] = act_ref[...].T

    out_ref[...] = jnp.dot(q_ref[...], at_ref[...],
                           preferred_element_type=jnp.float32)


def _score_call(queries, active):
    tile = 1024
    grid = _QN // tile
    nact = _NUM_HASHES * _ROW_LEN
    return pl.pallas_call(
        _score_body,
        grid=(grid,),
        in_specs=[
            pl.BlockSpec((tile, _D), lambda i: (i, 0)),
            pl.BlockSpec((nact, _D), lambda i: (0, 0)),
        ],
        out_specs=pl.BlockSpec((tile, nact), lambda i: (i, 0)),
        out_shape=jax.ShapeDtypeStruct((_QN, nact), jnp.float32),
        scratch_shapes=[pltpu.VMEM((_D, nact), jnp.float32)],
    )(queries, active)


def kernel(queries, kernels, a):
    at = jnp.pad(a.T, ((0, 0), (0, 4)))
    w = jnp.asarray(_W_np[:64])
    codes = _codes_call(queries, kernels, at, w)

    qcT = codes[:_QN, :_NUM_HASHES].T
    kcT = codes[_QN:, :_NUM_HASHES].T
    active = _sc_vote_gather(qcT, kcT, kernels)

    return _score_call(queries, active)


# speedup vs baseline: 1.1246x; 1.1246x over previous
import functools

import jax
import jax.numpy as jnp
import numpy as np
from jax import lax
from jax.experimental import pallas as pl
from jax.experimental.pallas import tpu as pltpu
from jax.experimental.pallas import tpu_sc as plsc

_NUM_HASHES = 5
_BITS = 12
_TABLE = 2048
_M = 3
_U = 0.83
_ROW_LEN = 64
_D = 512
_K = 4096
_QN = 8192
_N = _QN + _K
_HPAD = 8

_W_np = np.zeros((_NUM_HASHES * _BITS + 4, _HPAD), np.float32)
for _h in range(_NUM_HASHES):
    for _j in range(_BITS):
        _W_np[_h * _BITS + _j, _h] = float(2 ** _j) if _j < 11 else 0.0


_QT = _QN // 512
_KT = _K // 512


def _pack(x, at_ref, w_ref, out_ref):
    proj = jnp.dot(x, at_ref[...], preferred_element_type=jnp.float32)
    bits = (proj > 0).astype(jnp.float32)
    codes = jnp.dot(bits, w_ref[...], preferred_element_type=jnp.float32)
    out_ref[...] = codes.T


def _codes_body(q_ref, k_ref, at_ref, w_ref, out_ref, maxsq_ref):
    i = pl.program_id(0)

    @pl.when(i < _KT)
    def _():
        k = k_ref[...]
        mx = jnp.max(jnp.sum(k * k, axis=1))

        @pl.when(i == 0)
        def _():
            maxsq_ref[0] = mx

        @pl.when(i > 0)
        def _():
            maxsq_ref[0] = jnp.maximum(maxsq_ref[0], mx)

    @pl.when((i >= _KT) & (i < 2 * _KT))
    def _():
        scale = _U / jnp.sqrt(maxsq_ref[0])
        ku = k_ref[...] * scale
        sq = jnp.sum(ku * ku, axis=1, keepdims=True)
        s2 = sq * sq
        s4 = s2 * s2
        x = jnp.concatenate([ku, sq, s2, s4], axis=1)
        _pack(x, at_ref, w_ref, out_ref)

    @pl.when(i >= 2 * _KT)
    def _():
        q = q_ref[...]
        nrm = jnp.sqrt(jnp.sum(q * q, axis=1, keepdims=True))
        qn = q / (nrm + 1e-8)
        half = jnp.full((q.shape[0], _M), 0.5, jnp.float32)
        x = jnp.concatenate([qn, half], axis=1)
        _pack(x, at_ref, w_ref, out_ref)


def _codes_call(queries, kernels, at, w):
    def qmap(i):
        return (lax.max(i - 2 * _KT, 0), 0)

    def kmap(i):
        return (jnp.clip(jnp.where(i < _KT, i, i - _KT), 0, _KT - 1), 0)

    def omap(i):
        return (0, jnp.where(i < _KT, _QT,
                jnp.where(i < 2 * _KT, _QT + (i - _KT), i - 2 * _KT)))

    return pl.pallas_call(
        _codes_body,
        grid=(2 * _KT + _QT,),
        in_specs=[
            pl.BlockSpec((512, _D), qmap),
            pl.BlockSpec((512, _D), kmap),
            pl.BlockSpec((_D + _M, 64), lambda i: (0, 0)),
            pl.BlockSpec((64, _HPAD), lambda i: (0, 0)),
        ],
        out_specs=pl.BlockSpec((_HPAD, 512), omap),
        out_shape=jax.ShapeDtypeStruct((_HPAD, _N), jnp.float32),
        scratch_shapes=[pltpu.SMEM((1,), jnp.float32)],
    )(queries, kernels, at, w)


def _vote_body(qc_ref, kc3_ref, rows_ref):
    h = pl.program_id(0)
    qc = qc_ref[...]
    sel = (lax.broadcasted_iota(jnp.int32, (1, _HPAD), 1) == h).astype(jnp.float32)
    qcol = jnp.sum(qc * sel, axis=1, keepdims=True)
    hi = jnp.floor(qcol * (1.0 / 128.0))
    lo = qcol - hi * 128.0
    i16 = lax.broadcasted_iota(jnp.int32, (1, 16), 1).astype(jnp.float32)
    i128 = lax.broadcasted_iota(jnp.int32, (1, 128), 1).astype(jnp.float32)
    ehi = (hi == i16).astype(jnp.float32)
    elo = (lo == i128).astype(jnp.float32)
    counts = lax.dot_general(ehi, elo, (((0,), (0,)), ((), ())),
                             preferred_element_type=jnp.float32)
    maxv = jnp.max(counts)
    tids = (lax.broadcasted_iota(jnp.int32, (16, 128), 0) * 128
            + lax.broadcasted_iota(jnp.int32, (16, 128), 1)).astype(jnp.float32)
    best = jnp.min(jnp.where(counts == maxv, tids, 4096.0))
    kcv = kc3_ref[0]
    match = (kcv == best).astype(jnp.float32)
    tri = (lax.broadcasted_iota(jnp.int32, (128, 128), 0)
           <= lax.broadcasted_iota(jnp.int32, (128, 128), 1)).astype(jnp.float32)
    cumj = lax.dot_general(match, tri, (((1,), (0,)), ((), ())),
                           preferred_element_type=jnp.float32)
    ltri = (lax.broadcasted_iota(jnp.int32, (32, 32), 1)
            < lax.broadcasted_iota(jnp.int32, (32, 32), 0)).astype(jnp.float32)
    p = lax.dot_general(ltri, match, (((1,), (0,)), ((), ())),
                        preferred_element_type=jnp.float32)
    offs = jnp.sum(p, axis=1, keepdims=True)
    cum2 = cumj + offs
    iota64 = lax.broadcasted_iota(jnp.int32, (1, _ROW_LEN), 1)
    rows_f = jnp.zeros((1, _ROW_LEN), jnp.float32)
    for s in range(_ROW_LEN):
        cnt = jnp.sum((cum2 <= float(s)).astype(jnp.float32))
        rows_f = rows_f + cnt * (iota64 == s).astype(jnp.float32)
    rows_ref[0] = rows_f.astype(jnp.int32)


def _vote_call(qc, kc3):
    return pl.pallas_call(
        _vote_body,
        grid=(_NUM_HASHES,),
        in_specs=[
            pl.BlockSpec((_QN, _HPAD), lambda h: (0, 0)),
            pl.BlockSpec((1, 32, 128), lambda h: (h, 0, 0)),
        ],
        out_specs=pl.BlockSpec((1, 1, _ROW_LEN), lambda h: (h, 0, 0)),
        out_shape=jax.ShapeDtypeStruct((_NUM_HASHES, 1, _ROW_LEN), jnp.int32),
    )(qc, kc3)


def _sc_vote_gather(codesT, kernels):
    mesh = plsc.VectorSubcoreMesh(core_axis_name="c", subcore_axis_name="s")
    nact = _NUM_HASHES * _ROW_LEN

    @functools.partial(
        pl.kernel,
        out_type=jax.ShapeDtypeStruct((nact, _D), jnp.float32),
        mesh=mesh,
        compiler_params=pltpu.CompilerParams(needs_layout_passes=False),
        scratch_types=[
            pltpu.VMEM((6144,), jnp.int32),
            pltpu.VMEM((1536,), jnp.float32),
            pltpu.VMEM((_K,), jnp.float32),
            pltpu.VMEM((16, 128), jnp.int32),
            pltpu.VMEM((16,), jnp.int32),
            pltpu.VMEM((256,), jnp.int32),
            pltpu.VMEM((_ROW_LEN,), jnp.int32),
            pltpu.VMEM((_ROW_LEN, _D), jnp.float32),
            pltpu.VMEM_SHARED((16, 6144), jnp.int32),
            pltpu.VMEM_SHARED((256,), jnp.int32),
            pltpu.SemaphoreType.DMA,
        ],
    )
    def body(qcT_hbm, kern_hbm, active_hbm,
             hist_v, qbuf, kbuf, abuf, pbuf, bbuf, rows_v, rbuf,
             hist_sh, best_sh, sem):
        cid = lax.axis_index("c")
        sid = lax.axis_index("s")
        nh = jnp.where(cid == 0, 3, 2)
        hbase = jnp.where(cid == 0, 0, 3)

        z16 = jnp.zeros((16,), jnp.int32)
        zf16 = jnp.zeros((16,), jnp.float32)

        def zb(r, c):
            hist_v[pl.ds(r * 16, 16)] = z16
            return c
        lax.fori_loop(0, 384, zb, 0, unroll=8)

        def ql(j, c):
            pltpu.sync_copy(qcT_hbm.at[hbase + j, pl.ds(sid * 512, 512)],
                            qbuf.at[pl.ds(j * 512, 512)])
            return c
        lax.fori_loop(0, nh, ql, 0)

        ones16 = jnp.ones((16,), jnp.int32)

        def hu(t, c):
            @pl.when(t < nh * 32)
            def _():
                j = t // 32
                code = qbuf[pl.ds(t * 16, 16)]
                v = code.astype(jnp.int32) + j * 2048
                plsc.addupdate_scatter(hist_v, [v], ones16)
            return c
        lax.fori_loop(0, 96, hu, 0, unroll=4)

        pltpu.sync_copy(hist_v, hist_sh.at[sid])
        plsc.subcore_barrier()

        lane = lax.iota(jnp.int32, 16)

        def amh(j, pub):
            pltpu.sync_copy(
                hist_sh.at[:, pl.ds(j * 2048 + sid * 128, 128)], abuf)

            def am(r, carry):
                bv, bi = carry

                def acc_t(t, a):
                    return a + abuf[t, pl.ds(r * 16, 16)]
                cv = lax.fori_loop(1, 16, acc_t, abuf[0, pl.ds(r * 16, 16)],
                                   unroll=16)
                m = jnp.max(cv)
                f = plsc.all_reduce_ffs(cv == jnp.full((16,), m, jnp.int32))
                fi = jnp.max(f)
                upd = m > bv
                return (jnp.where(upd, m, bv),
                        jnp.where(upd, sid * 128 + r * 16 + fi, bi))
            bv, bi = lax.fori_loop(0, 8, am, (jnp.int32(-1), jnp.int32(0)))
            pub = jnp.where(lane == 2 * j, jnp.full((16,), bv, jnp.int32), pub)
            pub = jnp.where(lane == 2 * j + 1,
                            jnp.full((16,), bi, jnp.int32), pub)
            return pub
        pub = lax.fori_loop(0, nh, amh, z16)
        pbuf[...] = pub
        pltpu.sync_copy(pbuf, best_sh.at[pl.ds(sid * 16, 16)])
        plsc.subcore_barrier()

        @pl.when(sid < nh)
        def _():
            h = hbase + sid
            pltpu.sync_copy(best_sh, bbuf)
            neg = jnp.full((16,), -(2 ** 31) + 1, jnp.int32)

            def rd(t, carry):
                bv, bi = carry
                row = bbuf[pl.ds(t * 16, 16)]
                v = jnp.max(jnp.where(lane == 2 * sid, row, neg))
                i = jnp.max(jnp.where(lane == 2 * sid + 1, row, neg))
                upd = v > bv
                return (jnp.where(upd, v, bv), jnp.where(upd, i, bi))
            _, besti = lax.fori_loop(0, 16, rd,
                                     (jnp.int32(-1), jnp.int32(0)), unroll=16)

            pltpu.sync_copy(qcT_hbm.at[h, pl.ds(_QN, _K)], kbuf)

            def rz(r, c):
                rows_v[pl.ds(r * 16, 16)] = z16
                return c
            lax.fori_loop(0, _ROW_LEN // 16, rz, 0)

            bvec = jnp.full((16,), besti, jnp.int32)

            def cp(t, cnt0):
                code = kbuf[pl.ds(t * 16, 16)].astype(jnp.int32)
                m = code == bvec
                nm = jnp.max(plsc.all_reduce_population_count(m))

                @pl.when(nm > 0)
                def _():
                    cum = plsc.cumsum(m.astype(jnp.int32))
                    pos = cum + (cnt0 - 1)
                    wm = m & (pos < _ROW_LEN)
                    posc = jnp.clip(pos, 0, _ROW_LEN - 1)
                    rowid = lax.iota(jnp.int32, 16) + t * 16
                    plsc.store_scatter(rows_v, [posc], rowid, mask=wm)
                return cnt0 + nm
            total = lax.fori_loop(0, _K // 16, cp, jnp.int32(0), unroll=2)

            pltpu.async_copy(kern_hbm.at[rows_v], rbuf, sem).wait()

            def zz(s, c):
                @pl.when(s >= total)
                def _():
                    def zc(q, cc):
                        rbuf[s, pl.ds(q * 16, 16)] = zf16
                        return cc
                    lax.fori_loop(0, _D // 16, zc, 0, unroll=8)
                return c
            lax.fori_loop(0, _ROW_LEN, zz, 0)
            pltpu.sync_copy(rbuf, active_hbm.at[pl.ds(h * _ROW_LEN, _ROW_LEN)])

    return body(codesT, kernels)


def _score_body(q_ref, act_ref, out_ref, at_ref):
    @pl.when(pl.program_id(0) == 0)
    def _():
        at_ref[...] = act_ref[...].T

    out_ref[...] = jnp.dot(q_ref[...], at_ref[...],
                           preferred_element_type=jnp.float32)


def _score_call(queries, active):
    tile = 1024
    grid = _QN // tile
    nact = _NUM_HASHES * _ROW_LEN
    return pl.pallas_call(
        _score_body,
        grid=(grid,),
        in_specs=[
            pl.BlockSpec((tile, _D), lambda i: (i, 0)),
            pl.BlockSpec((nact, _D), lambda i: (0, 0)),
        ],
        out_specs=pl.BlockSpec((tile, nact), lambda i: (i, 0)),
        out_shape=jax.ShapeDtypeStruct((_QN, nact), jnp.float32),
        scratch_shapes=[pltpu.VMEM((_D, nact), jnp.float32)],
    )(queries, active)


def kernel(queries, kernels, a):
    at = jnp.pad(a.T, ((0, 0), (0, 4)))
    w = jnp.asarray(_W_np[:64])
    codesT = _codes_call(queries, kernels, at, w)

    active = _sc_vote_gather(codesT, kernels)

    return _score_call(queries, active)


# speedup vs baseline: 1.1270x; 1.0021x over previous
import functools

import jax
import jax.numpy as jnp
import numpy as np
from jax import lax
from jax.experimental import pallas as pl
from jax.experimental.pallas import tpu as pltpu
from jax.experimental.pallas import tpu_sc as plsc

_NUM_HASHES = 5
_BITS = 12
_TABLE = 2048
_M = 3
_U = 0.83
_ROW_LEN = 64
_D = 512
_K = 4096
_QN = 8192
_N = _QN + _K
_HPAD = 8

_W_np = np.zeros((_NUM_HASHES * _BITS + 4, _HPAD), np.float32)
for _h in range(_NUM_HASHES):
    for _j in range(_BITS):
        _W_np[_h * _BITS + _j, _h] = float(2 ** _j) if _j < 11 else 0.0


_QT = _QN // 512
_KT = _K // 512


def _pack(x, at_ref, w_ref, out_ref):
    proj = jnp.dot(x, at_ref[...], preferred_element_type=jnp.float32)
    bits = (proj > 0).astype(jnp.float32)
    codes = jnp.dot(bits, w_ref[...], preferred_element_type=jnp.float32)
    out_ref[...] = codes.T


def _codes_body(q_ref, k_ref, at_ref, w_ref, out_ref, maxsq_ref):
    i = pl.program_id(0)

    @pl.when(i < _KT)
    def _():
        k = k_ref[...]
        mx = jnp.max(jnp.sum(k * k, axis=1))

        @pl.when(i == 0)
        def _():
            maxsq_ref[0] = mx

        @pl.when(i > 0)
        def _():
            maxsq_ref[0] = jnp.maximum(maxsq_ref[0], mx)

    @pl.when((i >= _KT) & (i < 2 * _KT))
    def _():
        scale = _U / jnp.sqrt(maxsq_ref[0])
        ku = k_ref[...] * scale
        sq = jnp.sum(ku * ku, axis=1, keepdims=True)
        s2 = sq * sq
        s4 = s2 * s2
        x = jnp.concatenate([ku, sq, s2, s4], axis=1)
        _pack(x, at_ref, w_ref, out_ref)

    @pl.when(i >= 2 * _KT)
    def _():
        q = q_ref[...]
        nrm = jnp.sqrt(jnp.sum(q * q, axis=1, keepdims=True))
        qn = q / (nrm + 1e-8)
        half = jnp.full((q.shape[0], _M), 0.5, jnp.float32)
        x = jnp.concatenate([qn, half], axis=1)
        _pack(x, at_ref, w_ref, out_ref)


def _codes_call(queries, kernels, at, w):
    def qmap(i):
        return (lax.max(i - 2 * _KT, 0), 0)

    def kmap(i):
        return (jnp.clip(jnp.where(i < _KT, i, i - _KT), 0, _KT - 1), 0)

    def omap(i):
        return (0, jnp.where(i < _KT, _QT,
                jnp.where(i < 2 * _KT, _QT + (i - _KT), i - 2 * _KT)))

    return pl.pallas_call(
        _codes_body,
        grid=(2 * _KT + _QT,),
        in_specs=[
            pl.BlockSpec((512, _D), qmap),
            pl.BlockSpec((512, _D), kmap),
            pl.BlockSpec((_D + _M, 64), lambda i: (0, 0)),
            pl.BlockSpec((64, _HPAD), lambda i: (0, 0)),
        ],
        out_specs=pl.BlockSpec((_HPAD, 512), omap),
        out_shape=jax.ShapeDtypeStruct((_HPAD, _N), jnp.float32),
        scratch_shapes=[pltpu.SMEM((1,), jnp.float32)],
    )(queries, kernels, at, w)


def _sc_vote_gather(codesT, kernels):
    mesh = plsc.VectorSubcoreMesh(core_axis_name="c", subcore_axis_name="s")
    nact = _NUM_HASHES * _ROW_LEN

    @functools.partial(
        pl.kernel,
        out_type=jax.ShapeDtypeStruct((nact, _D), jnp.float32),
        mesh=mesh,
        compiler_params=pltpu.CompilerParams(needs_layout_passes=False),
        scratch_types=[
            pltpu.VMEM((6144,), jnp.int32),
            pltpu.VMEM((1536,), jnp.float32),
            pltpu.VMEM((_K,), jnp.float32),
            pltpu.VMEM((16, 128), jnp.int32),
            pltpu.VMEM((16,), jnp.int32),
            pltpu.VMEM((256,), jnp.int32),
            pltpu.VMEM((_ROW_LEN,), jnp.int32),
            pltpu.VMEM((_ROW_LEN, _D), jnp.float32),
            pltpu.VMEM_SHARED((16, 6144), jnp.int32),
            pltpu.VMEM_SHARED((256,), jnp.int32),
            pltpu.SemaphoreType.DMA,
        ],
    )
    def body(qcT_hbm, kern_hbm, active_hbm,
             hist_v, qbuf, kbuf, abuf, pbuf, bbuf, rows_v, rbuf,
             hist_sh, best_sh, sem):
        cid = lax.axis_index("c")
        sid = lax.axis_index("s")
        nh = jnp.where(cid == 0, 3, 2)
        hbase = jnp.where(cid == 0, 0, 3)

        z16 = jnp.zeros((16,), jnp.int32)
        zf16 = jnp.zeros((16,), jnp.float32)

        def zb(r, c):
            hist_v[pl.ds(r * 16, 16)] = z16
            return c
        lax.fori_loop(0, 384, zb, 0, unroll=8)

        def ql(j, c):
            pltpu.sync_copy(qcT_hbm.at[hbase + j, pl.ds(sid * 512, 512)],
                            qbuf.at[pl.ds(j * 512, 512)])
            return c
        lax.fori_loop(0, nh, ql, 0)

        ones16 = jnp.ones((16,), jnp.int32)

        def hu(t, c):
            @pl.when(t < nh * 32)
            def _():
                j = t // 32
                code = qbuf[pl.ds(t * 16, 16)]
                v = code.astype(jnp.int32) + j * 2048
                plsc.addupdate_scatter(hist_v, [v], ones16)
            return c
        lax.fori_loop(0, 96, hu, 0, unroll=4)

        pltpu.sync_copy(hist_v, hist_sh.at[sid])
        plsc.subcore_barrier()

        lane = lax.iota(jnp.int32, 16)

        def amh(j, pub):
            pltpu.sync_copy(
                hist_sh.at[:, pl.ds(j * 2048 + sid * 128, 128)], abuf)

            def am(r, carry):
                bv, bi = carry

                def acc_t(t, a):
                    return a + abuf[t, pl.ds(r * 16, 16)]
                cv = lax.fori_loop(1, 16, acc_t, abuf[0, pl.ds(r * 16, 16)],
                                   unroll=16)
                m = jnp.max(cv)
                f = plsc.all_reduce_ffs(cv == jnp.full((16,), m, jnp.int32))
                fi = jnp.max(f)
                upd = m > bv
                return (jnp.where(upd, m, bv),
                        jnp.where(upd, sid * 128 + r * 16 + fi, bi))
            bv, bi = lax.fori_loop(0, 8, am, (jnp.int32(-1), jnp.int32(0)))
            pub = jnp.where(lane == 2 * j, jnp.full((16,), bv, jnp.int32), pub)
            pub = jnp.where(lane == 2 * j + 1,
                            jnp.full((16,), bi, jnp.int32), pub)
            return pub
        pub = lax.fori_loop(0, nh, amh, z16)
        pbuf[...] = pub
        pltpu.sync_copy(pbuf, best_sh.at[pl.ds(sid * 16, 16)])
        plsc.subcore_barrier()

        @pl.when(sid < nh)
        def _():
            h = hbase + sid
            pltpu.sync_copy(best_sh, bbuf)
            neg = jnp.full((16,), -(2 ** 31) + 1, jnp.int32)

            def rd(t, carry):
                bv, bi = carry
                row = bbuf[pl.ds(t * 16, 16)]
                v = jnp.max(jnp.where(lane == 2 * sid, row, neg))
                i = jnp.max(jnp.where(lane == 2 * sid + 1, row, neg))
                upd = v > bv
                return (jnp.where(upd, v, bv), jnp.where(upd, i, bi))
            _, besti = lax.fori_loop(0, 16, rd,
                                     (jnp.int32(-1), jnp.int32(0)), unroll=16)

            pltpu.sync_copy(qcT_hbm.at[h, pl.ds(_QN, _K)], kbuf)

            def rz(r, c):
                rows_v[pl.ds(r * 16, 16)] = z16
                return c
            lax.fori_loop(0, _ROW_LEN // 16, rz, 0)

            bvec = jnp.full((16,), besti, jnp.int32)

            def cp(t, cnt0):
                code = kbuf[pl.ds(t * 16, 16)].astype(jnp.int32)
                m = code == bvec
                nm = jnp.max(plsc.all_reduce_population_count(m))

                @pl.when(nm > 0)
                def _():
                    cum = plsc.cumsum(m.astype(jnp.int32))
                    pos = cum + (cnt0 - 1)
                    wm = m & (pos < _ROW_LEN)
                    posc = jnp.clip(pos, 0, _ROW_LEN - 1)
                    rowid = lax.iota(jnp.int32, 16) + t * 16
                    plsc.store_scatter(rows_v, [posc], rowid, mask=wm)
                return cnt0 + nm
            total = lax.fori_loop(0, _K // 16, cp, jnp.int32(0), unroll=2)

            pltpu.async_copy(kern_hbm.at[rows_v], rbuf, sem).wait()

            def zz(s, c):
                @pl.when(s >= total)
                def _():
                    def zc(q, cc):
                        rbuf[s, pl.ds(q * 16, 16)] = zf16
                        return cc
                    lax.fori_loop(0, _D // 16, zc, 0, unroll=8)
                return c
            lax.fori_loop(0, _ROW_LEN, zz, 0)
            pltpu.sync_copy(rbuf, active_hbm.at[pl.ds(h * _ROW_LEN, _ROW_LEN)])

    return body(codesT, kernels)


def _score_body(q_ref, act_ref, out_ref, at_ref):
    @pl.when(pl.program_id(0) == 0)
    def _():
        at_ref[...] = act_ref[...].T

    out_ref[...] = jnp.dot(q_ref[...], at_ref[...],
                           preferred_element_type=jnp.float32)


def _score_call(queries, active):
    tile = 1024
    grid = _QN // tile
    nact = _NUM_HASHES * _ROW_LEN
    return pl.pallas_call(
        _score_body,
        grid=(grid,),
        in_specs=[
            pl.BlockSpec((tile, _D), lambda i: (i, 0)),
            pl.BlockSpec((nact, _D), lambda i: (0, 0)),
        ],
        out_specs=pl.BlockSpec((tile, nact), lambda i: (i, 0)),
        out_shape=jax.ShapeDtypeStruct((_QN, nact), jnp.float32),
        scratch_shapes=[pltpu.VMEM((_D, nact), jnp.float32)],
    )(queries, active)


def kernel(queries, kernels, a):
    at = jnp.pad(a.T, ((0, 0), (0, 4)))
    w = jnp.asarray(_W_np[:64])
    codesT = _codes_call(queries, kernels, at, w)

    active = _sc_vote_gather(codesT, kernels)

    return _score_call(queries, active)
